# baseline (speedup 1.0000x reference)
import functools

import jax
import jax.numpy as jnp
from jax import lax
from jax.experimental import pallas as pl
from jax.experimental.pallas import tpu as pltpu
from jax.experimental.pallas import tpu_sc as plsc

N = 320000
D = 128
NSEG = 10000

NC = 2
NS = 16
NW = NC * NS
BATCH = 128
NB = N // BATCH
KROWS = 2560
NP = KROWS * BATCH
RPW = KROWS // NW
NBLAST = NB - (NW - 1) * RPW
CH = 4
NCHUNK = RPW // CH
ZCH = 640

_mesh = plsc.VectorSubcoreMesh(core_axis_name="c", subcore_axis_name="s")


@functools.partial(
    pl.kernel,
    mesh=_mesh,
    out_type=jax.ShapeDtypeStruct((NC, NSEG), jnp.float32),
    scratch_types=[
        pltpu.VMEM((RPW, BATCH), jnp.int32),
        pltpu.VMEM((RPW, BATCH), jnp.int32),
        pltpu.VMEM((RPW, BATCH), jnp.float32),
        pltpu.VMEM((ZCH,), jnp.float32),
        pltpu.VMEM_SHARED((NSEG,), jnp.float32),
        pltpu.SemaphoreType.DMA,
        pltpu.SemaphoreType.DMA,
        pltpu.SemaphoreType.DMA,
        pltpu.SemaphoreType.DMA,
    ],
)
def _segsum_sc(src_hbm, idx_hbm, part_hbm,
               idx_v, gidx_v, vbuf, zbuf, acc, semA, semB, isem, ssem):
    c = lax.axis_index("c")
    s = lax.axis_index("s")
    wid = c * NS + s
    row0 = wid * RPW
    last = wid == NW - 1
    nb = jnp.where(last, NBLAST, RPW)

    pltpu.async_copy(idx_hbm.at[pl.ds(row0, RPW)], idx_v, isem)

    iota = lax.iota(jnp.int32, 16)
    b0 = row0 * (BATCH * D)
    lane = iota * D

    def gen(j, carry):
        base = b0 + j * (BATCH * D)
        for m in range(8):
            gidx_v[j, pl.ds(16 * m, 16)] = base + 16 * D * m + lane
        return carry

    lax.fori_loop(0, RPW, gen, 0)

    def fire_chunk(t, sem):
        for k in range(CH):
            j = t * CH + k

            @pl.when(j < nb)
            def _():
                pltpu.async_copy(src_hbm.at[gidx_v.at[j]], vbuf.at[j], sem)

    def drain_chunk(t, sem):
        for k in range(CH):
            j = t * CH + k

            @pl.when(j < nb)
            def _():
                pltpu.make_async_copy(
                    src_hbm.at[gidx_v.at[j]], vbuf.at[j], sem).wait()

    def scat_chunk(t):
        for k in range(CH):
            j = t * CH + k

            @pl.when(j < nb)
            def _():
                pltpu.async_copy(vbuf.at[j], acc.at[idx_v.at[j]], ssem,
                                 add=True)

    fire_chunk(0, semA)

    zv = jnp.zeros((16,), jnp.float32)

    def zstep(i, carry):
        zbuf[pl.ds(16 * i, 16)] = zv
        return carry

    lax.fori_loop(0, ZCH // 16, zstep, 0)

    @pl.when(s < NS - 1)
    def _():
        pltpu.sync_copy(zbuf, acc.at[pl.ds(s * ZCH, ZCH)])

    @pl.when(s == NS - 1)
    def _():
        pltpu.sync_copy(zbuf.at[pl.ds(0, NSEG - (NS - 1) * ZCH)],
                        acc.at[pl.ds((NS - 1) * ZCH, NSEG - (NS - 1) * ZCH)])

    pltpu.make_async_copy(idx_hbm.at[pl.ds(row0, RPW)], idx_v, isem).wait()
    plsc.subcore_barrier()

    def body(u, carry):
        t0 = 2 * u
        fire_chunk(t0 + 1, semB)
        drain_chunk(t0, semA)
        scat_chunk(t0)
        fire_chunk(t0 + 2, semA)
        drain_chunk(t0 + 1, semB)
        scat_chunk(t0 + 1)
        return carry

    lax.fori_loop(0, NCHUNK // 2, body, 0)

    def sdrain(j, carry):
        @pl.when(j < nb)
        def _():
            pltpu.make_async_copy(vbuf.at[j], acc.at[idx_v.at[j]],
                                  ssem).wait()
        return carry

    lax.fori_loop(0, RPW, sdrain, 0)
    plsc.subcore_barrier()

    @pl.when(s == 0)
    def _():
        pltpu.sync_copy(acc, part_hbm.at[c])


_RB = 1000


def _assemble_body(p_ref, o_ref):
    p = p_ref[...]
    total = p[0, 0] + p[0, 1]
    colid = lax.broadcasted_iota(jnp.int32, (_RB, D), 1)
    o_ref[...] = jnp.where(colid == 0, total[:, None], 0.0)


_assemble = pl.pallas_call(
    _assemble_body,
    grid=(NSEG // _RB,),
    in_specs=[pl.BlockSpec((1, NC, _RB), lambda i: (i, 0, 0))],
    out_specs=pl.BlockSpec((_RB, D), lambda i: (i, 0)),
    out_shape=jax.ShapeDtypeStruct((NSEG, D), jnp.float32),
)


def kernel(src, index):
    srcf = src.reshape(N * D)
    idx = jnp.pad(index, (0, NP - N)).reshape(KROWS, BATCH)
    part = _segsum_sc(srcf, idx)
    p3 = part.reshape(NC, NSEG // _RB, _RB).transpose(1, 0, 2)
    return _assemble(p3)


# speedup vs baseline: 1.1122x; 1.1122x over previous
import functools

import jax
import jax.numpy as jnp
from jax import lax
from jax.experimental import pallas as pl
from jax.experimental.pallas import tpu as pltpu
from jax.experimental.pallas import tpu_sc as plsc

N = 320000
D = 128
NSEG = 10000

NC = 2
NS = 16
NW = NC * NS
BATCH = 128
NB = N // BATCH
KROWS = 2560
NP = KROWS * BATCH
RPW = KROWS // NW
NBLAST = NB - (NW - 1) * RPW
CH = 4
NCHUNK = RPW // CH
ZCH = 640

_mesh = plsc.VectorSubcoreMesh(core_axis_name="c", subcore_axis_name="s")


@functools.partial(
    pl.kernel,
    mesh=_mesh,
    out_type=jax.ShapeDtypeStruct((NC, NSEG), jnp.float32),
    scratch_types=[
        pltpu.VMEM((RPW, BATCH), jnp.int32),
        pltpu.VMEM((RPW, BATCH), jnp.int32),
        pltpu.VMEM((RPW, BATCH), jnp.float32),
        pltpu.VMEM((ZCH,), jnp.float32),
        pltpu.VMEM_SHARED((NSEG,), jnp.float32),
        pltpu.SemaphoreType.DMA,
        pltpu.SemaphoreType.DMA,
        pltpu.SemaphoreType.DMA,
        pltpu.SemaphoreType.DMA,
    ],
)
def _segsum_sc(src_hbm, idx_hbm, part_hbm,
               idx_v, gidx_v, vbuf, zbuf, acc, semA, semB, isem, ssem):
    c = lax.axis_index("c")
    s = lax.axis_index("s")
    wid = c * NS + s
    row0 = wid * RPW
    last = wid == NW - 1
    nb = jnp.where(last, NBLAST, RPW)

    pltpu.async_copy(idx_hbm.at[pl.ds(row0, RPW)], idx_v, isem)

    iota = lax.iota(jnp.int32, 16)
    b0 = row0 * (BATCH * D)
    lane = iota * D

    def gen(j, carry):
        base = b0 + j * (BATCH * D)
        for m in range(8):
            gidx_v[j, pl.ds(16 * m, 16)] = base + 16 * D * m + lane
        return carry

    lax.fori_loop(0, RPW, gen, 0)

    def fire_chunk(t, sem):
        for k in range(CH):
            j = t * CH + k

            @pl.when(j < nb)
            def _():
                pltpu.async_copy(src_hbm.at[gidx_v.at[j]], vbuf.at[j], sem)

    def drain_chunk(t, sem):
        for k in range(CH):
            j = t * CH + k

            @pl.when(j < nb)
            def _():
                pltpu.make_async_copy(
                    src_hbm.at[gidx_v.at[j]], vbuf.at[j], sem).wait()

    def scat_chunk(t):
        for k in range(CH):
            j = t * CH + k

            @pl.when(j < nb)
            def _():
                pltpu.async_copy(vbuf.at[j], acc.at[idx_v.at[j]], ssem,
                                 add=True)

    fire_chunk(0, semA)

    zv = jnp.zeros((16,), jnp.float32)

    def zstep(i, carry):
        zbuf[pl.ds(16 * i, 16)] = zv
        return carry

    lax.fori_loop(0, ZCH // 16, zstep, 0)

    @pl.when(s < NS - 1)
    def _():
        pltpu.sync_copy(zbuf, acc.at[pl.ds(s * ZCH, ZCH)])

    @pl.when(s == NS - 1)
    def _():
        pltpu.sync_copy(zbuf.at[pl.ds(0, NSEG - (NS - 1) * ZCH)],
                        acc.at[pl.ds((NS - 1) * ZCH, NSEG - (NS - 1) * ZCH)])

    pltpu.make_async_copy(idx_hbm.at[pl.ds(row0, RPW)], idx_v, isem).wait()
    plsc.subcore_barrier()

    def body(u, carry):
        t0 = 2 * u
        fire_chunk(t0 + 1, semB)
        drain_chunk(t0, semA)
        scat_chunk(t0)
        fire_chunk(t0 + 2, semA)
        drain_chunk(t0 + 1, semB)
        scat_chunk(t0 + 1)
        return carry

    lax.fori_loop(0, NCHUNK // 2, body, 0)

    def sdrain(j, carry):
        @pl.when(j < nb)
        def _():
            pltpu.make_async_copy(vbuf.at[j], acc.at[idx_v.at[j]],
                                  ssem).wait()
        return carry

    lax.fori_loop(0, RPW, sdrain, 0)
    plsc.subcore_barrier()

    @pl.when(s == 0)
    def _():
        pltpu.sync_copy(acc, part_hbm.at[c])


_RB = 1000


def _assemble_body(p_ref, o_ref):
    p = p_ref[...]
    total = p[0] + p[1]
    colid = lax.broadcasted_iota(jnp.int32, (NSEG, D), 1)
    o_ref[...] = jnp.where(colid == 0, total[:, None], 0.0)


_assemble = pl.pallas_call(
    _assemble_body,
    out_shape=jax.ShapeDtypeStruct((NSEG, D), jnp.float32),
)


def kernel(src, index):
    srcf = src.reshape(N * D)
    idx = jnp.pad(index, (0, NP - N)).reshape(KROWS, BATCH)
    part = _segsum_sc(srcf, idx)
    return _assemble(part)
